# Initial kernel scaffold; baseline (speedup 1.0000x reference)
import jax
import jax.numpy as jnp
from jax.experimental import pallas as pl


def kernel(x, edge_index, W1, att_src1, att_dst1, b1, W2, att_src2, att_dst2, b2):
    raise NotImplementedError("write your pallas kernel here")


# speedup vs baseline: 12.2566x; 12.2566x over previous
import functools

import jax
import jax.numpy as jnp
from jax import lax
from jax.experimental import pallas as pl
from jax.experimental.pallas import tpu as pltpu
from jax.experimental.pallas import tpu_sc as plsc

N = 10000
IN_CH = 128
HID = 128
OUT = 64
E = 320000

NPAD = 10240
NC, NS = 2, 16
NW = NC * NS
ETOT = E + N
EBLK = 1296
NBLK = 8
EPW = EBLK * NBLK
EPAD = EPW * NW
CH = 16
D = 128

_EPS = 1e-16


_R = 1024


def _lin1_body(x_ref, w_ref, batt_ref, h_ref, aux_ref):
    h = jnp.dot(x_ref[...], w_ref[...], preferred_element_type=jnp.float32)
    h_ref[...] = h
    aux_ref[...] = jnp.dot(h, batt_ref[...], preferred_element_type=jnp.float32)


def _lin1(xpad, W1, Batt1):
    return pl.pallas_call(
        _lin1_body,
        grid=(NPAD // _R,),
        in_specs=[
            pl.BlockSpec((_R, IN_CH), lambda i: (i, 0)),
            pl.BlockSpec((IN_CH, HID), lambda i: (0, 0)),
            pl.BlockSpec((HID, 128), lambda i: (0, 0)),
        ],
        out_specs=[
            pl.BlockSpec((_R, HID), lambda i: (i, 0)),
            pl.BlockSpec((_R, 128), lambda i: (i, 0)),
        ],
        out_shape=[
            jax.ShapeDtypeStruct((NPAD, HID), jnp.float32),
            jax.ShapeDtypeStruct((NPAD, 128), jnp.float32),
        ],
    )(xpad, W1, Batt1)


def _lin2_body(p_ref, den_ref, b1_ref, w2_ref, batt_ref, h2_ref, aux_ref):
    den = jnp.sum(den_ref[...], axis=0)
    t = (p_ref[0] + p_ref[1]) / (den + _EPS)
    t = t + b1_ref[...]
    t = jnp.where(t > 0, t, jnp.exp(jnp.minimum(t, 0.0)) - 1.0)
    h2 = jnp.dot(t, w2_ref[...], preferred_element_type=jnp.float32)
    h2_ref[...] = h2
    aux_ref[...] = jnp.dot(h2, batt_ref[...], preferred_element_type=jnp.float32)


def _lin2(P1, den1, b1, W2pad, Batt2):
    return pl.pallas_call(
        _lin2_body,
        grid=(NPAD // _R,),
        in_specs=[
            pl.BlockSpec((2, _R, D), lambda i: (0, i, 0)),
            pl.BlockSpec((NW, _R, 1), lambda i: (0, i, 0)),
            pl.BlockSpec((1, HID), lambda i: (0, 0)),
            pl.BlockSpec((HID, 128), lambda i: (0, 0)),
            pl.BlockSpec((128, 128), lambda i: (0, 0)),
        ],
        out_specs=[
            pl.BlockSpec((_R, 128), lambda i: (i, 0)),
            pl.BlockSpec((_R, 128), lambda i: (i, 0)),
        ],
        out_shape=[
            jax.ShapeDtypeStruct((NPAD, 128), jnp.float32),
            jax.ShapeDtypeStruct((NPAD, 128), jnp.float32),
        ],
    )(P1, den1, b1.reshape(1, HID), W2pad, Batt2)


def _final_body(q_ref, den_ref, b2_ref, out_ref):
    den = jnp.sum(den_ref[...], axis=0)
    v = (q_ref[0, :, :OUT] + q_ref[1, :, :OUT]) / (den + _EPS)
    v = v + b2_ref[...]
    m = jnp.max(v, axis=1, keepdims=True)
    v = v - m
    out_ref[...] = v - jnp.log(jnp.sum(jnp.exp(v), axis=1, keepdims=True))


def _final(P2, den2, b2):
    return pl.pallas_call(
        _final_body,
        grid=(NPAD // _R,),
        in_specs=[
            pl.BlockSpec((2, _R, D), lambda i: (0, i, 0)),
            pl.BlockSpec((NW, _R, 1), lambda i: (0, i, 0)),
            pl.BlockSpec((1, OUT), lambda i: (0, 0)),
        ],
        out_specs=pl.BlockSpec((_R, OUT), lambda i: (i, 0)),
        out_shape=jax.ShapeDtypeStruct((NPAD, OUT), jnp.float32),
    )(P2, den2, b2.reshape(1, OUT))


def _make_edge_kernel():
    ZCH = 16
    RPT = NPAD // NS

    mesh = plsc.VectorSubcoreMesh(core_axis_name="c", subcore_axis_name="s")

    @functools.partial(
        pl.kernel,
        out_type=(
            jax.ShapeDtypeStruct((NC, NPAD, D), jnp.float32),
            jax.ShapeDtypeStruct((NW, NPAD), jnp.float32),
        ),
        mesh=mesh,
        compiler_params=pltpu.CompilerParams(needs_layout_passes=False),
        scratch_types=[
            pltpu.VMEM((EBLK,), jnp.int32),
            pltpu.VMEM((EBLK,), jnp.int32),
            pltpu.VMEM((NPAD,), jnp.float32),
            pltpu.VMEM((NPAD,), jnp.float32),
            pltpu.VMEM((NPAD + 16,), jnp.float32),
            pltpu.VMEM((32,), jnp.float32),
            pltpu.VMEM((16,), jnp.int32),
            pltpu.VMEM((16,), jnp.int32),
            pltpu.VMEM((16, D), jnp.float32),
            pltpu.VMEM((16, D), jnp.float32),
            pltpu.VMEM((ZCH, D), jnp.float32),
            pltpu.VMEM_SHARED((NPAD, D), jnp.float32),
            pltpu.SemaphoreType.DMA,
        ],
    )
    def edge_kernel(src_hbm, dst_hbm, asrc_hbm, adst_hbm, h_hbm,
                    out_hbm, den_hbm,
                    src_v, dst_v, asrc_v, adst_v, den_v, wbuf, gidx, sidx,
                    rows, rows_sc, zbuf, acc, sem):
        c = lax.axis_index("c")
        s = lax.axis_index("s")
        wid = s * NC + c

        def zden_body(i, _):
            den_v[pl.ds(i * 16, 16)] = jnp.zeros((16,), jnp.float32)
            return 0

        lax.fori_loop(0, (NPAD + 16) // 16, zden_body, 0)

        def zero_body(i, _):
            for j in range(D // 16):
                zbuf[i, pl.ds(j * 16, 16)] = jnp.zeros((16,), jnp.float32)
            return 0

        lax.fori_loop(0, ZCH, zero_body, 0)
        for rep in range(RPT // ZCH):
            pltpu.sync_copy(zbuf, acc.at[pl.ds(s * RPT + rep * ZCH, ZCH)])
        plsc.subcore_barrier()

        pltpu.sync_copy(asrc_hbm, asrc_v)
        pltpu.sync_copy(adst_hbm, adst_v)

        base = wid * EPW
        lane = lax.iota(jnp.int32, 16)

        def edge_body(g, _):
            off = g * CH
            src16 = src_v[pl.ds(off, CH)]
            dst16 = dst_v[pl.ds(off, CH)]
            asv = plsc.load_gather(asrc_v, [src16])
            adv = plsc.load_gather(adst_v, [dst16])
            z = asv + adv
            e = jnp.maximum(z, 0.2 * z)
            w = jnp.exp(e)
            wbuf[pl.ds(0, 16)] = w
            wbuf[pl.ds(16, 16)] = w
            gidx[...] = src16
            sidx[...] = dst16
            for l in range(16):
                idx_l = jnp.where(lane == l, dst16, NPAD + lane)
                plsc.addupdate_scatter(den_v, [idx_l], w)
            pltpu.async_copy(h_hbm.at[gidx], rows, sem).wait()
            for r in range(16):
                wb = plsc.load_gather(wbuf, [jnp.full((16,), 16 + r, jnp.int32)])
                for j in range(D // 16):
                    rows_sc[r, pl.ds(j * 16, 16)] = (
                        rows[r, pl.ds(j * 16, 16)] * wb)
            pltpu.sync_copy(rows_sc, acc.at[sidx], add=True)
            return 0

        def blk_body(bi, _):
            bbase = base + bi * EBLK
            pltpu.sync_copy(src_hbm.at[pl.ds(bbase, EBLK)], src_v)
            pltpu.sync_copy(dst_hbm.at[pl.ds(bbase, EBLK)], dst_v)
            lax.fori_loop(0, EBLK // CH, edge_body, 0)
            return 0

        lax.fori_loop(0, NBLK, blk_body, 0)

        pltpu.sync_copy(den_v.at[pl.ds(0, NPAD)], den_hbm.at[wid])
        plsc.subcore_barrier()
        pltpu.sync_copy(acc.at[pl.ds(s * RPT, RPT)],
                        out_hbm.at[c, pl.ds(s * RPT, RPT)])

    return edge_kernel


_edge = _make_edge_kernel()


def kernel(x, edge_index, W1, att_src1, att_dst1, b1, W2, att_src2, att_dst2, b2):
    f32 = jnp.float32
    xpad = jnp.zeros((NPAD, IN_CH), f32).at[:N].set(x)

    Batt1 = jnp.zeros((HID, 128), f32)
    Batt1 = Batt1.at[:, 0].set(att_src1[0]).at[:, 1].set(att_dst1[0])
    Batt2 = jnp.zeros((128, 128), f32)
    Batt2 = Batt2.at[:OUT, 0].set(att_src2[0]).at[:OUT, 1].set(att_dst2[0])
    W2pad = jnp.zeros((HID, 128), f32).at[:, :OUT].set(W2)

    loop = jnp.arange(N, dtype=jnp.int32)
    padv = jnp.full((EPAD - ETOT,), N, dtype=jnp.int32)
    src = jnp.concatenate([edge_index[0], loop, padv])
    dst = jnp.concatenate([edge_index[1], loop, padv])

    h1, aux1 = _lin1(xpad, W1, Batt1)
    P1, den1 = _edge(src, dst, aux1[:, 0], aux1[:, 1], h1)
    h2, aux2 = _lin2(P1, den1.reshape(NW, NPAD, 1), b1, W2pad, Batt2)
    P2, den2 = _edge(src, dst, aux2[:, 0], aux2[:, 1], h2)
    out = _final(P2, den2.reshape(NW, NPAD, 1), b2)
    return out[:N]


# speedup vs baseline: 20.4289x; 1.6668x over previous
import functools

import jax
import jax.numpy as jnp
from jax import lax
from jax.experimental import pallas as pl
from jax.experimental.pallas import tpu as pltpu
from jax.experimental.pallas import tpu_sc as plsc

N = 10000
IN_CH = 128
HID = 128
OUT = 64
E = 320000

NPAD = 10240
NC, NS = 2, 16
NW = NC * NS
ETOT = E + N
EBLK = 1296
NBLK = 8
EPW = EBLK * NBLK
EPAD = EPW * NW
CH = 16
D = 128

_EPS = 1e-16


_R = 1024


def _lin1_body(x_ref, w_ref, batt_ref, h_ref, aux_ref):
    h = jnp.dot(x_ref[...], w_ref[...], preferred_element_type=jnp.float32)
    h_ref[...] = h
    aux_ref[...] = jnp.dot(h, batt_ref[...], preferred_element_type=jnp.float32)


def _lin1(xpad, W1, Batt1):
    return pl.pallas_call(
        _lin1_body,
        grid=(NPAD // _R,),
        in_specs=[
            pl.BlockSpec((_R, IN_CH), lambda i: (i, 0)),
            pl.BlockSpec((IN_CH, HID), lambda i: (0, 0)),
            pl.BlockSpec((HID, 128), lambda i: (0, 0)),
        ],
        out_specs=[
            pl.BlockSpec((_R, HID), lambda i: (i, 0)),
            pl.BlockSpec((_R, 128), lambda i: (i, 0)),
        ],
        out_shape=[
            jax.ShapeDtypeStruct((NPAD, HID), jnp.float32),
            jax.ShapeDtypeStruct((NPAD, 128), jnp.float32),
        ],
    )(xpad, W1, Batt1)


def _lin2_body(p_ref, den_ref, b1_ref, w2_ref, batt_ref, h2_ref, aux_ref):
    den = jnp.sum(den_ref[...], axis=0)
    t = (p_ref[0] + p_ref[1]) / (den + _EPS)
    t = t + b1_ref[...]
    t = jnp.where(t > 0, t, jnp.exp(jnp.minimum(t, 0.0)) - 1.0)
    h2 = jnp.dot(t, w2_ref[...], preferred_element_type=jnp.float32)
    h2_ref[...] = h2
    aux_ref[...] = jnp.dot(h2, batt_ref[...], preferred_element_type=jnp.float32)


def _lin2(P1, den1, b1, W2pad, Batt2):
    return pl.pallas_call(
        _lin2_body,
        grid=(NPAD // _R,),
        in_specs=[
            pl.BlockSpec((2, _R, D), lambda i: (0, i, 0)),
            pl.BlockSpec((NW, _R, 1), lambda i: (0, i, 0)),
            pl.BlockSpec((1, HID), lambda i: (0, 0)),
            pl.BlockSpec((HID, 128), lambda i: (0, 0)),
            pl.BlockSpec((128, 128), lambda i: (0, 0)),
        ],
        out_specs=[
            pl.BlockSpec((_R, 128), lambda i: (i, 0)),
            pl.BlockSpec((_R, 128), lambda i: (i, 0)),
        ],
        out_shape=[
            jax.ShapeDtypeStruct((NPAD, 128), jnp.float32),
            jax.ShapeDtypeStruct((NPAD, 128), jnp.float32),
        ],
    )(P1, den1, b1.reshape(1, HID), W2pad, Batt2)


def _final_body(q_ref, den_ref, b2_ref, out_ref):
    den = jnp.sum(den_ref[...], axis=0)
    v = (q_ref[0, :, :OUT] + q_ref[1, :, :OUT]) / (den + _EPS)
    v = v + b2_ref[...]
    m = jnp.max(v, axis=1, keepdims=True)
    v = v - m
    out_ref[...] = v - jnp.log(jnp.sum(jnp.exp(v), axis=1, keepdims=True))


def _final(P2, den2, b2):
    return pl.pallas_call(
        _final_body,
        grid=(NPAD // _R,),
        in_specs=[
            pl.BlockSpec((2, _R, D), lambda i: (0, i, 0)),
            pl.BlockSpec((NW, _R, 1), lambda i: (0, i, 0)),
            pl.BlockSpec((1, OUT), lambda i: (0, 0)),
        ],
        out_specs=pl.BlockSpec((_R, OUT), lambda i: (i, 0)),
        out_shape=jax.ShapeDtypeStruct((NPAD, OUT), jnp.float32),
    )(P2, den2, b2.reshape(1, OUT))


def _make_edge_kernel():
    ZCH = 16
    RPT = NPAD // NS

    mesh = plsc.VectorSubcoreMesh(core_axis_name="c", subcore_axis_name="s")

    @functools.partial(
        pl.kernel,
        out_type=(
            jax.ShapeDtypeStruct((NC, NPAD, D), jnp.float32),
            jax.ShapeDtypeStruct((NW, NPAD), jnp.float32),
        ),
        mesh=mesh,
        compiler_params=pltpu.CompilerParams(needs_layout_passes=False),
        scratch_types=[
            pltpu.VMEM((EBLK,), jnp.int32),
            pltpu.VMEM((EBLK,), jnp.int32),
            pltpu.VMEM((NPAD,), jnp.float32),
            pltpu.VMEM((NPAD,), jnp.float32),
            pltpu.VMEM((NPAD + 16,), jnp.float32),
            pltpu.VMEM((32,), jnp.float32),
            pltpu.VMEM((32,), jnp.float32),
            pltpu.VMEM((16,), jnp.int32),
            pltpu.VMEM((16,), jnp.int32),
            pltpu.VMEM((16,), jnp.int32),
            pltpu.VMEM((16,), jnp.int32),
            pltpu.VMEM((16, D), jnp.float32),
            pltpu.VMEM((16, D), jnp.float32),
            pltpu.VMEM((16, D), jnp.float32),
            pltpu.VMEM((16, D), jnp.float32),
            pltpu.VMEM((ZCH, D), jnp.float32),
            pltpu.VMEM_SHARED((NPAD, D), jnp.float32),
            pltpu.SemaphoreType.DMA,
            pltpu.SemaphoreType.DMA,
            pltpu.SemaphoreType.DMA,
            pltpu.SemaphoreType.DMA,
        ],
    )
    def edge_kernel(src_hbm, dst_hbm, asrc_hbm, adst_hbm, h_hbm,
                    out_hbm, den_hbm,
                    src_v, dst_v, asrc_v, adst_v, den_v, wbuf0, wbuf1,
                    gidx0, gidx1, sidx0, sidx1, rows0, rows1, rsc0, rsc1,
                    zbuf, acc, sg0, sg1, ss0, ss1):
        c = lax.axis_index("c")
        s = lax.axis_index("s")
        wid = s * NC + c

        def zden_body(i, _):
            den_v[pl.ds(i * 16, 16)] = jnp.zeros((16,), jnp.float32)
            return 0

        lax.fori_loop(0, (NPAD + 16) // 16, zden_body, 0)

        def zero_body(i, _):
            for j in range(D // 16):
                zbuf[i, pl.ds(j * 16, 16)] = jnp.zeros((16,), jnp.float32)
            return 0

        lax.fori_loop(0, ZCH, zero_body, 0)
        for rep in range(RPT // ZCH):
            pltpu.sync_copy(zbuf, acc.at[pl.ds(s * RPT + rep * ZCH, ZCH)])
        plsc.subcore_barrier()

        pltpu.sync_copy(asrc_hbm, asrc_v)
        pltpu.sync_copy(adst_hbm, adst_v)

        base = wid * EPW
        lane = lax.iota(jnp.int32, 16)
        NCHB = EBLK // CH
        wbuf = (wbuf0, wbuf1)
        gidx = (gidx0, gidx1)
        sidx = (sidx0, sidx1)
        rows = (rows0, rows1)
        rsc = (rsc0, rsc1)
        sg = (sg0, sg1)
        ss = (ss0, ss1)

        def process(g, b):
            bn = 1 - b
            off = g * CH
            src16 = src_v[pl.ds(off, CH)]
            dst16 = dst_v[pl.ds(off, CH)]
            asv = plsc.load_gather(asrc_v, [src16])
            adv = plsc.load_gather(adst_v, [dst16])
            z = asv + adv
            e = jnp.maximum(z, 0.2 * z)
            w = jnp.exp(e)
            wbuf[b][pl.ds(0, 16)] = w
            wbuf[b][pl.ds(16, 16)] = w
            sidx[b][...] = dst16

            @pl.when(g + 1 < NCHB)
            def _():
                gidx[bn][...] = src_v[pl.ds((g + 1) * CH, CH)]
                pltpu.async_copy(h_hbm.at[gidx[bn]], rows[bn], sg[bn])

            for l in range(16):
                idx_l = jnp.where(lane == l, dst16, NPAD + lane)
                plsc.addupdate_scatter(den_v, [idx_l], w)

            pltpu.make_async_copy(h_hbm.at[gidx[b]], rows[b], sg[b]).wait()
            @pl.when(g >= 2)
            def _():
                pltpu.make_async_copy(h_hbm.at[gidx[b]], rsc[b], ss[b]).wait()
            for r in range(16):
                wb = plsc.load_gather(
                    wbuf[b], [jnp.full((16,), 16 + r, jnp.int32)])
                for j in range(D // 16):
                    rsc[b][r, pl.ds(j * 16, 16)] = (
                        rows[b][r, pl.ds(j * 16, 16)] * wb)
            pltpu.async_copy(rsc[b], acc.at[sidx[b]], ss[b], add=True)

        def pair_body(g2, _):
            for b in (0, 1):
                g = g2 * 2 + b

                @pl.when(g < NCHB)
                def _():
                    process(g, b)

            return 0

        def blk_body(bi, _):
            bbase = base + bi * EBLK
            pltpu.sync_copy(src_hbm.at[pl.ds(bbase, EBLK)], src_v)
            pltpu.sync_copy(dst_hbm.at[pl.ds(bbase, EBLK)], dst_v)
            gidx[0][...] = src_v[pl.ds(0, CH)]
            pltpu.async_copy(h_hbm.at[gidx[0]], rows[0], sg[0])
            lax.fori_loop(0, (NCHB + 1) // 2, pair_body, 0)
            pltpu.make_async_copy(h_hbm.at[gidx[0]], rsc0, ss0).wait()
            pltpu.make_async_copy(h_hbm.at[gidx[0]], rsc1, ss1).wait()
            return 0

        lax.fori_loop(0, NBLK, blk_body, 0)

        pltpu.sync_copy(den_v.at[pl.ds(0, NPAD)], den_hbm.at[wid])
        plsc.subcore_barrier()
        pltpu.sync_copy(acc.at[pl.ds(s * RPT, RPT)],
                        out_hbm.at[c, pl.ds(s * RPT, RPT)])

    return edge_kernel


_edge = _make_edge_kernel()


def kernel(x, edge_index, W1, att_src1, att_dst1, b1, W2, att_src2, att_dst2, b2):
    f32 = jnp.float32
    xpad = jnp.zeros((NPAD, IN_CH), f32).at[:N].set(x)

    Batt1 = jnp.zeros((HID, 128), f32)
    Batt1 = Batt1.at[:, 0].set(att_src1[0]).at[:, 1].set(att_dst1[0])
    Batt2 = jnp.zeros((128, 128), f32)
    Batt2 = Batt2.at[:OUT, 0].set(att_src2[0]).at[:OUT, 1].set(att_dst2[0])
    W2pad = jnp.zeros((HID, 128), f32).at[:, :OUT].set(W2)

    loop = jnp.arange(N, dtype=jnp.int32)
    padv = jnp.full((EPAD - ETOT,), N, dtype=jnp.int32)
    src = jnp.concatenate([edge_index[0], loop, padv])
    dst = jnp.concatenate([edge_index[1], loop, padv])

    h1, aux1 = _lin1(xpad, W1, Batt1)
    P1, den1 = _edge(src, dst, aux1[:, 0], aux1[:, 1], h1)
    h2, aux2 = _lin2(P1, den1.reshape(NW, NPAD, 1), b1, W2pad, Batt2)
    P2, den2 = _edge(src, dst, aux2[:, 0], aux2[:, 1], h2)
    out = _final(P2, den2.reshape(NW, NPAD, 1), b2)
    return out[:N]


# speedup vs baseline: 25.0429x; 1.2259x over previous
import functools

import jax
import jax.numpy as jnp
from jax import lax
from jax.experimental import pallas as pl
from jax.experimental.pallas import tpu as pltpu
from jax.experimental.pallas import tpu_sc as plsc

N = 10000
IN_CH = 128
HID = 128
OUT = 64
E = 320000

NPAD = 10240
NC, NS = 2, 16
NW = NC * NS
ETOT = E + N
EBLK = 1296
NBLK = 8
EPW = EBLK * NBLK
EPAD = EPW * NW
CH = 16
D = 128

_EPS = 1e-16


_R = 1024


def _lin1_body(x_ref, w_ref, batt_ref, h_ref, aux_ref):
    h = jnp.dot(x_ref[...], w_ref[...], preferred_element_type=jnp.float32)
    h_ref[...] = h
    aux_ref[...] = jnp.dot(h, batt_ref[...], preferred_element_type=jnp.float32)


def _lin1(xpad, W1, Batt1):
    return pl.pallas_call(
        _lin1_body,
        grid=(NPAD // _R,),
        in_specs=[
            pl.BlockSpec((_R, IN_CH), lambda i: (i, 0)),
            pl.BlockSpec((IN_CH, HID), lambda i: (0, 0)),
            pl.BlockSpec((HID, 128), lambda i: (0, 0)),
        ],
        out_specs=[
            pl.BlockSpec((_R, HID), lambda i: (i, 0)),
            pl.BlockSpec((_R, 128), lambda i: (i, 0)),
        ],
        out_shape=[
            jax.ShapeDtypeStruct((NPAD, HID), jnp.float32),
            jax.ShapeDtypeStruct((NPAD, 128), jnp.float32),
        ],
    )(xpad, W1, Batt1)


def _lin2_body(p_ref, den_ref, b1_ref, w2_ref, batt_ref, h2_ref, aux_ref):
    den = jnp.sum(den_ref[...], axis=0)
    t = (p_ref[0] + p_ref[1]) / (den + _EPS)
    t = t + b1_ref[...]
    t = jnp.where(t > 0, t, jnp.exp(jnp.minimum(t, 0.0)) - 1.0)
    h2 = jnp.dot(t, w2_ref[...], preferred_element_type=jnp.float32)
    h2_ref[...] = h2
    aux_ref[...] = jnp.dot(h2, batt_ref[...], preferred_element_type=jnp.float32)


def _lin2(P1, den1, b1, W2pad, Batt2):
    return pl.pallas_call(
        _lin2_body,
        grid=(NPAD // _R,),
        in_specs=[
            pl.BlockSpec((2, _R, D), lambda i: (0, i, 0)),
            pl.BlockSpec((NW, _R, 1), lambda i: (0, i, 0)),
            pl.BlockSpec((1, HID), lambda i: (0, 0)),
            pl.BlockSpec((HID, 128), lambda i: (0, 0)),
            pl.BlockSpec((128, 128), lambda i: (0, 0)),
        ],
        out_specs=[
            pl.BlockSpec((_R, 128), lambda i: (i, 0)),
            pl.BlockSpec((_R, 128), lambda i: (i, 0)),
        ],
        out_shape=[
            jax.ShapeDtypeStruct((NPAD, 128), jnp.float32),
            jax.ShapeDtypeStruct((NPAD, 128), jnp.float32),
        ],
    )(P1, den1, b1.reshape(1, HID), W2pad, Batt2)


def _final_body(q_ref, den_ref, b2_ref, out_ref):
    den = jnp.sum(den_ref[...], axis=0)
    v = (q_ref[0, :, :OUT] + q_ref[1, :, :OUT]) / (den + _EPS)
    v = v + b2_ref[...]
    m = jnp.max(v, axis=1, keepdims=True)
    v = v - m
    out_ref[...] = v - jnp.log(jnp.sum(jnp.exp(v), axis=1, keepdims=True))


def _final(P2, den2, b2):
    return pl.pallas_call(
        _final_body,
        grid=(NPAD // _R,),
        in_specs=[
            pl.BlockSpec((2, _R, D), lambda i: (0, i, 0)),
            pl.BlockSpec((NW, _R, 1), lambda i: (0, i, 0)),
            pl.BlockSpec((1, OUT), lambda i: (0, 0)),
        ],
        out_specs=pl.BlockSpec((_R, OUT), lambda i: (i, 0)),
        out_shape=jax.ShapeDtypeStruct((NPAD, OUT), jnp.float32),
    )(P2, den2, b2.reshape(1, OUT))


def _make_edge_kernel():
    ZCH = 8
    RPT = NPAD // NS

    mesh = plsc.VectorSubcoreMesh(core_axis_name="c", subcore_axis_name="s")

    @functools.partial(
        pl.kernel,
        out_type=(
            jax.ShapeDtypeStruct((NC, NPAD, D), jnp.float32),
            jax.ShapeDtypeStruct((NW, NPAD), jnp.float32),
        ),
        mesh=mesh,
        compiler_params=pltpu.CompilerParams(needs_layout_passes=False),
        scratch_types=[
            pltpu.VMEM((EBLK,), jnp.int32),
            pltpu.VMEM((EBLK,), jnp.int32),
            pltpu.VMEM((NPAD,), jnp.float32),
            pltpu.VMEM((NPAD,), jnp.float32),
            pltpu.VMEM((NPAD + 16,), jnp.float32),
            pltpu.VMEM((32,), jnp.float32),
            pltpu.VMEM((32,), jnp.float32),
            pltpu.VMEM((16,), jnp.int32),
            pltpu.VMEM((16,), jnp.int32),
            pltpu.VMEM((16,), jnp.int32),
            pltpu.VMEM((16,), jnp.int32),
            pltpu.VMEM((16,), jnp.int32),
            pltpu.VMEM((16,), jnp.int32),
            pltpu.VMEM((16, D), jnp.float32),
            pltpu.VMEM((16, D), jnp.float32),
            pltpu.VMEM((16, D), jnp.float32),
            pltpu.VMEM((16, D), jnp.float32),
            pltpu.VMEM((16, D), jnp.float32),
            pltpu.VMEM((16, D), jnp.float32),
            pltpu.VMEM((ZCH, D), jnp.float32),
            pltpu.VMEM_SHARED((NPAD, D), jnp.float32),
            pltpu.SemaphoreType.DMA,
            pltpu.SemaphoreType.DMA,
            pltpu.SemaphoreType.DMA,
            pltpu.SemaphoreType.DMA,
            pltpu.SemaphoreType.DMA,
            pltpu.SemaphoreType.DMA,
        ],
    )
    def edge_kernel(src_hbm, dst_hbm, asrc_hbm, adst_hbm, h_hbm,
                    out_hbm, den_hbm,
                    src_v, dst_v, asrc_v, adst_v, den_v, wbuf0, wbuf1,
                    gidx0, gidx1, gidx2, gidx3, sidx0, sidx1,
                    rows0, rows1, rows2, rows3, rsc0, rsc1,
                    zbuf, acc, sg0, sg1, sg2, sg3, ss0, ss1):
        c = lax.axis_index("c")
        s = lax.axis_index("s")
        wid = s * NC + c

        def zden_body(i, _):
            den_v[pl.ds(i * 16, 16)] = jnp.zeros((16,), jnp.float32)
            return 0

        lax.fori_loop(0, (NPAD + 16) // 16, zden_body, 0)

        def zero_body(i, _):
            for j in range(D // 16):
                zbuf[i, pl.ds(j * 16, 16)] = jnp.zeros((16,), jnp.float32)
            return 0

        lax.fori_loop(0, ZCH, zero_body, 0)
        for rep in range(RPT // ZCH):
            pltpu.async_copy(zbuf, acc.at[pl.ds(s * RPT + rep * ZCH, ZCH)],
                             sg0)
        for rep in range(RPT // ZCH):
            pltpu.make_async_copy(h_hbm.at[pl.ds(0, ZCH)], zbuf, sg0).wait()
        plsc.subcore_barrier()

        pltpu.sync_copy(asrc_hbm, asrc_v)
        pltpu.sync_copy(adst_hbm, adst_v)

        base = wid * EPW
        lane = lax.iota(jnp.int32, 16)
        NCHB = EBLK // CH
        GB = 4
        wbuf = (wbuf0, wbuf1)
        gidx = (gidx0, gidx1, gidx2, gidx3)
        sidx = (sidx0, sidx1)
        rows = (rows0, rows1, rows2, rows3)
        rsc = (rsc0, rsc1)
        sg = (sg0, sg1, sg2, sg3)
        ss = (ss0, ss1)

        def prefetch(g, gb):
            gidx[gb][...] = src_v[pl.ds(g * CH, CH)]
            pltpu.async_copy(h_hbm.at[gidx[gb]], rows[gb], sg[gb])

        def process(g, gb, sb):
            off = g * CH
            src16 = src_v[pl.ds(off, CH)]
            dst16 = dst_v[pl.ds(off, CH)]
            asv = plsc.load_gather(asrc_v, [src16])
            adv = plsc.load_gather(adst_v, [dst16])
            z = asv + adv
            e = jnp.maximum(z, 0.2 * z)
            w = jnp.exp(e)

            @pl.when(g + GB - 1 < NCHB)
            def _():
                prefetch(g + GB - 1, (gb + GB - 1) % GB)

            for l in range(16):
                idx_l = jnp.where(lane == l, dst16, NPAD + lane)
                plsc.addupdate_scatter(den_v, [idx_l], w)

            pltpu.make_async_copy(h_hbm.at[gidx[gb]], rows[gb], sg[gb]).wait()
            @pl.when(g >= 2)
            def _():
                pltpu.make_async_copy(h_hbm.at[gidx[gb]], rsc[sb], ss[sb]).wait()
            wbuf[sb][pl.ds(0, 16)] = w
            wbuf[sb][pl.ds(16, 16)] = w
            sidx[sb][...] = dst16
            for r in range(16):
                wb = plsc.load_gather(
                    wbuf[sb], [jnp.full((16,), 16 + r, jnp.int32)])
                for j in range(D // 16):
                    rsc[sb][r, pl.ds(j * 16, 16)] = (
                        rows[gb][r, pl.ds(j * 16, 16)] * wb)
            pltpu.async_copy(rsc[sb], acc.at[sidx[sb]], ss[sb], add=True)

        def quad_body(g4, _):
            for i in range(GB):
                g = g4 * GB + i

                @pl.when(g < NCHB)
                def _():
                    process(g, i, i % 2)

            return 0

        def blk_body(bi, _):
            bbase = base + bi * EBLK
            pltpu.sync_copy(src_hbm.at[pl.ds(bbase, EBLK)], src_v)
            pltpu.sync_copy(dst_hbm.at[pl.ds(bbase, EBLK)], dst_v)
            for g in range(GB - 1):
                prefetch(g, g)
            lax.fori_loop(0, (NCHB + GB - 1) // GB, quad_body, 0)
            pltpu.make_async_copy(h_hbm.at[gidx[0]], rsc0, ss0).wait()
            pltpu.make_async_copy(h_hbm.at[gidx[0]], rsc1, ss1).wait()
            return 0

        lax.fori_loop(0, NBLK, blk_body, 0)

        pltpu.sync_copy(den_v.at[pl.ds(0, NPAD)], den_hbm.at[wid])
        plsc.subcore_barrier()
        pltpu.sync_copy(acc.at[pl.ds(s * RPT, RPT)],
                        out_hbm.at[c, pl.ds(s * RPT, RPT)])

    return edge_kernel


_edge = _make_edge_kernel()


def kernel(x, edge_index, W1, att_src1, att_dst1, b1, W2, att_src2, att_dst2, b2):
    f32 = jnp.float32
    xpad = jnp.zeros((NPAD, IN_CH), f32).at[:N].set(x)

    Batt1 = jnp.zeros((HID, 128), f32)
    Batt1 = Batt1.at[:, 0].set(att_src1[0]).at[:, 1].set(att_dst1[0])
    Batt2 = jnp.zeros((128, 128), f32)
    Batt2 = Batt2.at[:OUT, 0].set(att_src2[0]).at[:OUT, 1].set(att_dst2[0])
    W2pad = jnp.zeros((HID, 128), f32).at[:, :OUT].set(W2)

    loop = jnp.arange(N, dtype=jnp.int32)
    padv = jnp.full((EPAD - ETOT,), N, dtype=jnp.int32)
    src = jnp.concatenate([edge_index[0], loop, padv])
    dst = jnp.concatenate([edge_index[1], loop, padv])

    h1, aux1 = _lin1(xpad, W1, Batt1)
    P1, den1 = _edge(src, dst, aux1[:, 0], aux1[:, 1], h1)
    h2, aux2 = _lin2(P1, den1.reshape(NW, NPAD, 1), b1, W2pad, Batt2)
    P2, den2 = _edge(src, dst, aux2[:, 0], aux2[:, 1], h2)
    out = _final(P2, den2.reshape(NW, NPAD, 1), b2)
    return out[:N]


# speedup vs baseline: 31.2603x; 1.2483x over previous
import functools

import jax
import jax.numpy as jnp
from jax import lax
from jax.experimental import pallas as pl
from jax.experimental.pallas import tpu as pltpu
from jax.experimental.pallas import tpu_sc as plsc

N = 10000
IN_CH = 128
HID = 128
OUT = 64
E = 320000

NPAD = 10240
NC, NS = 2, 16
NW = NC * NS
EBLK = 2000
NBLK = 5
EPW = EBLK * NBLK
CH = 16
D = 128
NDEN = 10112

_EPS = 1e-16


def _leaky(z):
    return jnp.maximum(z, 0.2 * z)


_R = 1000


def _lin1_body(x_ref, w_ref, batt_ref, h_ref, aux_ref):
    h = jnp.dot(x_ref[...], w_ref[...], preferred_element_type=jnp.float32)
    h_ref[...] = h
    aux_ref[...] = jnp.dot(h, batt_ref[...], preferred_element_type=jnp.float32)


def _lin1(x, W1, Batt1):
    return pl.pallas_call(
        _lin1_body,
        grid=(N // _R,),
        in_specs=[
            pl.BlockSpec((_R, IN_CH), lambda i: (i, 0)),
            pl.BlockSpec((IN_CH, HID), lambda i: (0, 0)),
            pl.BlockSpec((HID, 8), lambda i: (0, 0)),
        ],
        out_specs=[
            pl.BlockSpec((_R, HID), lambda i: (i, 0)),
            pl.BlockSpec((_R, 8), lambda i: (i, 0)),
        ],
        out_shape=[
            jax.ShapeDtypeStruct((N, HID), jnp.float32),
            jax.ShapeDtypeStruct((N, 8), jnp.float32),
        ],
    )(x, W1, Batt1)


def _lin2_body(p_ref, den_ref, aux_ref, h1_ref, b1_ref, w2_ref, batt_ref,
               h2_ref, aux2_ref):
    wself = jnp.exp(_leaky(aux_ref[:, 0:1] + aux_ref[:, 1:2]))
    den = jnp.sum(den_ref[...], axis=0) + wself + _EPS
    p = p_ref[0] + p_ref[1] + wself * h1_ref[...]
    t = p / den + b1_ref[...]
    t = jnp.where(t > 0, t, jnp.exp(jnp.minimum(t, 0.0)) - 1.0)
    h2 = jnp.dot(t, w2_ref[...], preferred_element_type=jnp.float32)
    h2_ref[...] = h2
    aux2_ref[...] = jnp.dot(h2, batt_ref[...], preferred_element_type=jnp.float32)


def _lin2(P1, den1, aux1, h1, b1, W2pad, Batt2):
    return pl.pallas_call(
        _lin2_body,
        grid=(N // _R,),
        in_specs=[
            pl.BlockSpec((2, _R, D), lambda i: (0, i, 0)),
            pl.BlockSpec((NW, _R, 1), lambda i: (0, i, 0)),
            pl.BlockSpec((_R, 8), lambda i: (i, 0)),
            pl.BlockSpec((_R, HID), lambda i: (i, 0)),
            pl.BlockSpec((1, HID), lambda i: (0, 0)),
            pl.BlockSpec((HID, 128), lambda i: (0, 0)),
            pl.BlockSpec((128, 8), lambda i: (0, 0)),
        ],
        out_specs=[
            pl.BlockSpec((_R, 128), lambda i: (i, 0)),
            pl.BlockSpec((_R, 8), lambda i: (i, 0)),
        ],
        out_shape=[
            jax.ShapeDtypeStruct((N, 128), jnp.float32),
            jax.ShapeDtypeStruct((N, 8), jnp.float32),
        ],
    )(P1, den1, aux1, h1, b1.reshape(1, HID), W2pad, Batt2)


def _final_body(q_ref, den_ref, aux_ref, h2_ref, b2_ref, out_ref):
    wself = jnp.exp(_leaky(aux_ref[:, 0:1] + aux_ref[:, 1:2]))
    den = jnp.sum(den_ref[...], axis=0) + wself + _EPS
    v = (q_ref[0, :, :OUT] + q_ref[1, :, :OUT]
         + wself * h2_ref[:, :OUT]) / den
    v = v + b2_ref[...]
    m = jnp.max(v, axis=1, keepdims=True)
    v = v - m
    out_ref[...] = v - jnp.log(jnp.sum(jnp.exp(v), axis=1, keepdims=True))


def _final(P2, den2, aux2, h2, b2):
    return pl.pallas_call(
        _final_body,
        grid=(N // _R,),
        in_specs=[
            pl.BlockSpec((2, _R, D), lambda i: (0, i, 0)),
            pl.BlockSpec((NW, _R, 1), lambda i: (0, i, 0)),
            pl.BlockSpec((_R, 8), lambda i: (i, 0)),
            pl.BlockSpec((_R, 128), lambda i: (i, 0)),
            pl.BlockSpec((1, OUT), lambda i: (0, 0)),
        ],
        out_specs=pl.BlockSpec((_R, OUT), lambda i: (i, 0)),
        out_shape=jax.ShapeDtypeStruct((N, OUT), jnp.float32),
    )(P2, den2, aux2, h2, b2.reshape(1, OUT))


def _make_edge_kernel():
    ZCH = 8
    RPT = NPAD // NS

    mesh = plsc.VectorSubcoreMesh(core_axis_name="c", subcore_axis_name="s")

    @functools.partial(
        pl.kernel,
        out_type=(
            jax.ShapeDtypeStruct((NC, NPAD, D), jnp.float32),
            jax.ShapeDtypeStruct((NW, NDEN), jnp.float32),
        ),
        mesh=mesh,
        compiler_params=pltpu.CompilerParams(needs_layout_passes=False),
        scratch_types=[
            pltpu.VMEM((EBLK,), jnp.int32),
            pltpu.VMEM((EBLK,), jnp.int32),
            pltpu.VMEM((N,), jnp.float32),
            pltpu.VMEM((N,), jnp.float32),
            pltpu.VMEM((NDEN,), jnp.float32),
            pltpu.VMEM((32,), jnp.float32),
            pltpu.VMEM((32,), jnp.float32),
            pltpu.VMEM((16,), jnp.int32),
            pltpu.VMEM((16,), jnp.int32),
            pltpu.VMEM((16,), jnp.int32),
            pltpu.VMEM((16,), jnp.int32),
            pltpu.VMEM((16,), jnp.int32),
            pltpu.VMEM((16,), jnp.int32),
            pltpu.VMEM((16, D), jnp.float32),
            pltpu.VMEM((16, D), jnp.float32),
            pltpu.VMEM((16, D), jnp.float32),
            pltpu.VMEM((16, D), jnp.float32),
            pltpu.VMEM((16, D), jnp.float32),
            pltpu.VMEM((16, D), jnp.float32),
            pltpu.VMEM((ZCH, D), jnp.float32),
            pltpu.VMEM_SHARED((NPAD, D), jnp.float32),
            pltpu.SemaphoreType.DMA,
            pltpu.SemaphoreType.DMA,
            pltpu.SemaphoreType.DMA,
            pltpu.SemaphoreType.DMA,
            pltpu.SemaphoreType.DMA,
            pltpu.SemaphoreType.DMA,
        ],
    )
    def edge_kernel(src_hbm, dst_hbm, asrc_hbm, adst_hbm, h_hbm,
                    out_hbm, den_hbm,
                    src_v, dst_v, asrc_v, adst_v, den_v, wbuf0, wbuf1,
                    gidx0, gidx1, gidx2, gidx3, sidx0, sidx1,
                    rows0, rows1, rows2, rows3, rsc0, rsc1,
                    zbuf, acc, sg0, sg1, sg2, sg3, ss0, ss1):
        c = lax.axis_index("c")
        s = lax.axis_index("s")
        wid = s * NC + c

        def zden_body(i, _):
            den_v[pl.ds(i * 16, 16)] = jnp.zeros((16,), jnp.float32)
            return 0

        lax.fori_loop(0, NDEN // 16, zden_body, 0)

        def zero_body(i, _):
            for j in range(D // 16):
                zbuf[i, pl.ds(j * 16, 16)] = jnp.zeros((16,), jnp.float32)
            return 0

        lax.fori_loop(0, ZCH, zero_body, 0)
        for rep in range(RPT // ZCH):
            pltpu.async_copy(zbuf, acc.at[pl.ds(s * RPT + rep * ZCH, ZCH)],
                             sg0)
        for rep in range(RPT // ZCH):
            pltpu.make_async_copy(h_hbm.at[pl.ds(0, ZCH)], zbuf, sg0).wait()
        plsc.subcore_barrier()

        pltpu.sync_copy(asrc_hbm, asrc_v)
        pltpu.sync_copy(adst_hbm, adst_v)

        base = wid * EPW
        lane = lax.iota(jnp.int32, 16)
        NCHB = EBLK // CH
        GB = 4
        wbuf = (wbuf0, wbuf1)
        gidx = (gidx0, gidx1, gidx2, gidx3)
        sidx = (sidx0, sidx1)
        rows = (rows0, rows1, rows2, rows3)
        rsc = (rsc0, rsc1)
        sg = (sg0, sg1, sg2, sg3)
        ss = (ss0, ss1)

        def prefetch(g, gb):
            gidx[gb][...] = src_v[pl.ds(g * CH, CH)]
            pltpu.async_copy(h_hbm.at[gidx[gb]], rows[gb], sg[gb])

        def process(g, gb, sb):
            off = g * CH
            src16 = src_v[pl.ds(off, CH)]
            dst16 = dst_v[pl.ds(off, CH)]
            asv = plsc.load_gather(asrc_v, [src16])
            adv = plsc.load_gather(adst_v, [dst16])
            w = jnp.exp(_leaky(asv + adv))

            @pl.when(g + GB - 1 < NCHB)
            def _():
                prefetch(g + GB - 1, (gb + GB - 1) % GB)

            for l in range(16):
                idx_l = jnp.where(lane == l, dst16, N + lane)
                plsc.addupdate_scatter(den_v, [idx_l], w)

            pltpu.make_async_copy(h_hbm.at[gidx[gb]], rows[gb], sg[gb]).wait()
            @pl.when(g >= 2)
            def _():
                pltpu.make_async_copy(h_hbm.at[gidx[gb]], rsc[sb], ss[sb]).wait()
            wbuf[sb][pl.ds(0, 16)] = w
            wbuf[sb][pl.ds(16, 16)] = w
            sidx[sb][...] = dst16
            for r in range(16):
                wb = plsc.load_gather(
                    wbuf[sb], [jnp.full((16,), 16 + r, jnp.int32)])
                for j in range(D // 16):
                    rsc[sb][r, pl.ds(j * 16, 16)] = (
                        rows[gb][r, pl.ds(j * 16, 16)] * wb)
            pltpu.async_copy(rsc[sb], acc.at[sidx[sb]], ss[sb], add=True)

        def quad_body(g4, _):
            for i in range(GB):
                g = g4 * GB + i

                @pl.when(g < NCHB)
                def _():
                    process(g, i, i % 2)

            return 0

        def blk_body(bi, _):
            bbase = base + bi * EBLK
            pltpu.sync_copy(src_hbm.at[pl.ds(bbase, EBLK)], src_v)
            pltpu.sync_copy(dst_hbm.at[pl.ds(bbase, EBLK)], dst_v)
            for g in range(GB - 1):
                prefetch(g, g)
            lax.fori_loop(0, (NCHB + GB - 1) // GB, quad_body, 0)
            pltpu.make_async_copy(h_hbm.at[gidx[0]], rsc0, ss0).wait()
            pltpu.make_async_copy(h_hbm.at[gidx[0]], rsc1, ss1).wait()
            return 0

        lax.fori_loop(0, NBLK, blk_body, 0)

        pltpu.sync_copy(den_v, den_hbm.at[wid])
        plsc.subcore_barrier()
        pltpu.sync_copy(acc.at[pl.ds(s * RPT, RPT)],
                        out_hbm.at[c, pl.ds(s * RPT, RPT)])

    return edge_kernel


_edge = _make_edge_kernel()


def kernel(x, edge_index, W1, att_src1, att_dst1, b1, W2, att_src2, att_dst2, b2):
    f32 = jnp.float32

    Batt1 = jnp.zeros((HID, 8), f32)
    Batt1 = Batt1.at[:, 0].set(att_src1[0]).at[:, 1].set(att_dst1[0])
    Batt2 = jnp.zeros((128, 8), f32)
    Batt2 = Batt2.at[:OUT, 0].set(att_src2[0]).at[:OUT, 1].set(att_dst2[0])
    W2pad = jnp.zeros((HID, 128), f32).at[:, :OUT].set(W2)

    src = edge_index[0]
    dst = edge_index[1]

    h1, aux1 = _lin1(x, W1, Batt1)
    P1, den1 = _edge(src, dst, aux1[:, 0], aux1[:, 1], h1)
    h2, aux2 = _lin2(P1, den1.reshape(NW, NDEN, 1), aux1, h1, b1, W2pad, Batt2)
    P2, den2 = _edge(src, dst, aux2[:, 0], aux2[:, 1], h2)
    return _final(P2, den2.reshape(NW, NDEN, 1), aux2, h2, b2)


# speedup vs baseline: 46.7073x; 1.4941x over previous
import functools

import jax
import jax.numpy as jnp
from jax import lax
from jax.experimental import pallas as pl
from jax.experimental.pallas import tpu as pltpu
from jax.experimental.pallas import tpu_sc as plsc

N = 10000
IN_CH = 128
HID = 128
OUT = 64
E = 320000

NPAD = 10240
NC, NS = 2, 16
NW = NC * NS
EBLK = 2000
NBLK = 5
EPW = EBLK * NBLK
CH = 16
D = 128
NDEN = 10112

_EPS = 1e-16


def _leaky(z):
    return jnp.maximum(z, 0.2 * z)


_R = 1000


def _lin1_body(x_ref, w_ref, batt_ref, h_ref, aux_ref):
    h = jnp.dot(x_ref[...], w_ref[...], preferred_element_type=jnp.float32)
    h_ref[...] = h
    aux_ref[...] = jnp.dot(h, batt_ref[...], preferred_element_type=jnp.float32)


def _lin1(x, W1, Batt1):
    return pl.pallas_call(
        _lin1_body,
        grid=(N // _R,),
        in_specs=[
            pl.BlockSpec((_R, IN_CH), lambda i: (i, 0)),
            pl.BlockSpec((IN_CH, HID), lambda i: (0, 0)),
            pl.BlockSpec((HID, 8), lambda i: (0, 0)),
        ],
        out_specs=[
            pl.BlockSpec((_R, HID), lambda i: (i, 0)),
            pl.BlockSpec((_R, 8), lambda i: (i, 0)),
        ],
        out_shape=[
            jax.ShapeDtypeStruct((N, HID), jnp.float32),
            jax.ShapeDtypeStruct((N, 8), jnp.float32),
        ],
    )(x, W1, Batt1)


def _lin2_body(p_ref, den_ref, aux_ref, h1_ref, b1_ref, w2_ref, batt_ref,
               h2_ref, aux2_ref):
    wself = jnp.exp(_leaky(aux_ref[:, 0:1] + aux_ref[:, 1:2]))
    den = jnp.sum(den_ref[...], axis=1, keepdims=True) + wself + _EPS
    p = p_ref[0] + p_ref[1] + wself * h1_ref[...]
    t = p / den + b1_ref[...]
    t = jnp.where(t > 0, t, jnp.exp(jnp.minimum(t, 0.0)) - 1.0)
    h2 = jnp.dot(t, w2_ref[...], preferred_element_type=jnp.float32)
    h2_ref[...] = h2
    aux2_ref[...] = jnp.dot(h2, batt_ref[...], preferred_element_type=jnp.float32)


def _lin2(P1, den1, aux1, h1, b1, W2pad, Batt2):
    return pl.pallas_call(
        _lin2_body,
        grid=(N // _R,),
        in_specs=[
            pl.BlockSpec((2, _R, D), lambda i: (0, i, 0)),
            pl.BlockSpec((_R, NW), lambda i: (i, 0)),
            pl.BlockSpec((_R, 8), lambda i: (i, 0)),
            pl.BlockSpec((_R, HID), lambda i: (i, 0)),
            pl.BlockSpec((1, HID), lambda i: (0, 0)),
            pl.BlockSpec((HID, 128), lambda i: (0, 0)),
            pl.BlockSpec((128, 8), lambda i: (0, 0)),
        ],
        out_specs=[
            pl.BlockSpec((_R, 128), lambda i: (i, 0)),
            pl.BlockSpec((_R, 8), lambda i: (i, 0)),
        ],
        out_shape=[
            jax.ShapeDtypeStruct((N, 128), jnp.float32),
            jax.ShapeDtypeStruct((N, 8), jnp.float32),
        ],
    )(P1, den1, aux1, h1, b1.reshape(1, HID), W2pad, Batt2)


def _final_body(q_ref, den_ref, aux_ref, h2_ref, b2_ref, out_ref):
    wself = jnp.exp(_leaky(aux_ref[:, 0:1] + aux_ref[:, 1:2]))
    den = jnp.sum(den_ref[...], axis=1, keepdims=True) + wself + _EPS
    v = (q_ref[0, :, :OUT] + q_ref[1, :, :OUT]
         + wself * h2_ref[:, :OUT]) / den
    v = v + b2_ref[...]
    m = jnp.max(v, axis=1, keepdims=True)
    v = v - m
    out_ref[...] = v - jnp.log(jnp.sum(jnp.exp(v), axis=1, keepdims=True))


def _final(P2, den2, aux2, h2, b2):
    return pl.pallas_call(
        _final_body,
        grid=(N // _R,),
        in_specs=[
            pl.BlockSpec((2, _R, D), lambda i: (0, i, 0)),
            pl.BlockSpec((_R, NW), lambda i: (i, 0)),
            pl.BlockSpec((_R, 8), lambda i: (i, 0)),
            pl.BlockSpec((_R, 128), lambda i: (i, 0)),
            pl.BlockSpec((1, OUT), lambda i: (0, 0)),
        ],
        out_specs=pl.BlockSpec((_R, OUT), lambda i: (i, 0)),
        out_shape=jax.ShapeDtypeStruct((N, OUT), jnp.float32),
    )(P2, den2, aux2, h2, b2.reshape(1, OUT))


def _make_edge_kernel():
    ZCH = 8
    RPT = NPAD // NS

    mesh = plsc.VectorSubcoreMesh(core_axis_name="c", subcore_axis_name="s")

    @functools.partial(
        pl.kernel,
        out_type=(
            jax.ShapeDtypeStruct((NC, NPAD, D), jnp.float32),
            jax.ShapeDtypeStruct((NW, NDEN), jnp.float32),
        ),
        mesh=mesh,
        compiler_params=pltpu.CompilerParams(needs_layout_passes=False),
        scratch_types=[
            pltpu.VMEM((EBLK,), jnp.int32),
            pltpu.VMEM((EBLK,), jnp.int32),
            pltpu.VMEM((N,), jnp.float32),
            pltpu.VMEM((N,), jnp.float32),
            pltpu.VMEM((NDEN,), jnp.float32),
            pltpu.VMEM((32,), jnp.float32),
            pltpu.VMEM((32,), jnp.float32),
            pltpu.VMEM((16,), jnp.int32),
            pltpu.VMEM((16,), jnp.int32),
            pltpu.VMEM((16,), jnp.int32),
            pltpu.VMEM((16,), jnp.int32),
            pltpu.VMEM((16,), jnp.int32),
            pltpu.VMEM((16,), jnp.int32),
            pltpu.VMEM((16, D), jnp.float32),
            pltpu.VMEM((16, D), jnp.float32),
            pltpu.VMEM((16, D), jnp.float32),
            pltpu.VMEM((16, D), jnp.float32),
            pltpu.VMEM((16, D), jnp.float32),
            pltpu.VMEM((16, D), jnp.float32),
            pltpu.VMEM((ZCH, D), jnp.float32),
            pltpu.VMEM_SHARED((NPAD, D), jnp.float32),
            pltpu.SemaphoreType.DMA,
            pltpu.SemaphoreType.DMA,
            pltpu.SemaphoreType.DMA,
            pltpu.SemaphoreType.DMA,
            pltpu.SemaphoreType.DMA,
            pltpu.SemaphoreType.DMA,
        ],
    )
    def edge_kernel(src_hbm, dst_hbm, asrc_hbm, adst_hbm, h_hbm,
                    out_hbm, den_hbm,
                    src_v, dst_v, asrc_v, adst_v, den_v, wbuf0, wbuf1,
                    gidx0, gidx1, gidx2, gidx3, sidx0, sidx1,
                    rows0, rows1, rows2, rows3, rsc0, rsc1,
                    zbuf, acc, sg0, sg1, sg2, sg3, ss0, ss1):
        c = lax.axis_index("c")
        s = lax.axis_index("s")
        wid = s * NC + c

        def zden_body(i, _):
            den_v[pl.ds(i * 16, 16)] = jnp.zeros((16,), jnp.float32)
            return 0

        lax.fori_loop(0, NDEN // 16, zden_body, 0)

        def zero_body(i, _):
            for j in range(D // 16):
                zbuf[i, pl.ds(j * 16, 16)] = jnp.zeros((16,), jnp.float32)
            return 0

        lax.fori_loop(0, ZCH, zero_body, 0)
        for rep in range(RPT // ZCH):
            pltpu.async_copy(zbuf, acc.at[pl.ds(s * RPT + rep * ZCH, ZCH)],
                             sg0)
        for rep in range(RPT // ZCH):
            pltpu.make_async_copy(h_hbm.at[pl.ds(0, ZCH)], zbuf, sg0).wait()
        plsc.subcore_barrier()

        pltpu.sync_copy(asrc_hbm, asrc_v)
        pltpu.sync_copy(adst_hbm, adst_v)

        base = wid * EPW
        lane = lax.iota(jnp.int32, 16)
        NCHB = EBLK // CH
        GB = 4
        wbuf = (wbuf0, wbuf1)
        gidx = (gidx0, gidx1, gidx2, gidx3)
        sidx = (sidx0, sidx1)
        rows = (rows0, rows1, rows2, rows3)
        rsc = (rsc0, rsc1)
        sg = (sg0, sg1, sg2, sg3)
        ss = (ss0, ss1)

        def prefetch(g, gb):
            gidx[gb][...] = src_v[pl.ds(g * CH, CH)]
            pltpu.async_copy(h_hbm.at[gidx[gb]], rows[gb], sg[gb])

        def process(g, gb, sb):
            off = g * CH
            src16 = src_v[pl.ds(off, CH)]
            dst16 = dst_v[pl.ds(off, CH)]
            asv = plsc.load_gather(asrc_v, [src16])
            adv = plsc.load_gather(adst_v, [dst16])
            w = jnp.exp(_leaky(asv + adv))

            @pl.when(g + GB - 1 < NCHB)
            def _():
                prefetch(g + GB - 1, (gb + GB - 1) % GB)

            for l in range(16):
                idx_l = jnp.where(lane == l, dst16, N + lane)
                plsc.addupdate_scatter(den_v, [idx_l], w)

            pltpu.make_async_copy(h_hbm.at[gidx[gb]], rows[gb], sg[gb]).wait()
            @pl.when(g >= 2)
            def _():
                pltpu.make_async_copy(h_hbm.at[gidx[gb]], rsc[sb], ss[sb]).wait()
            wbuf[sb][pl.ds(0, 16)] = w
            wbuf[sb][pl.ds(16, 16)] = w
            sidx[sb][...] = dst16
            for r in range(16):
                wb = plsc.load_gather(
                    wbuf[sb], [jnp.full((16,), 16 + r, jnp.int32)])
                for j in range(D // 16):
                    rsc[sb][r, pl.ds(j * 16, 16)] = (
                        rows[gb][r, pl.ds(j * 16, 16)] * wb)
            pltpu.async_copy(rsc[sb], acc.at[sidx[sb]], ss[sb], add=True)

        def quad_body(g4, _):
            for i in range(GB):
                g = g4 * GB + i

                @pl.when(g < NCHB)
                def _():
                    process(g, i, i % 2)

            return 0

        def blk_body(bi, _):
            bbase = base + bi * EBLK
            pltpu.sync_copy(src_hbm.at[pl.ds(bbase, EBLK)], src_v)
            pltpu.sync_copy(dst_hbm.at[pl.ds(bbase, EBLK)], dst_v)
            for g in range(GB - 1):
                prefetch(g, g)
            lax.fori_loop(0, (NCHB + GB - 1) // GB, quad_body, 0)
            pltpu.make_async_copy(h_hbm.at[gidx[0]], rsc0, ss0).wait()
            pltpu.make_async_copy(h_hbm.at[gidx[0]], rsc1, ss1).wait()
            return 0

        lax.fori_loop(0, NBLK, blk_body, 0)

        pltpu.sync_copy(den_v, den_hbm.at[wid])
        plsc.subcore_barrier()
        pltpu.sync_copy(acc.at[pl.ds(s * RPT, RPT)],
                        out_hbm.at[c, pl.ds(s * RPT, RPT)])

    return edge_kernel


_edge = _make_edge_kernel()


def kernel(x, edge_index, W1, att_src1, att_dst1, b1, W2, att_src2, att_dst2, b2):
    f32 = jnp.float32

    Batt1 = jnp.zeros((HID, 8), f32)
    Batt1 = Batt1.at[:, 0].set(att_src1[0]).at[:, 1].set(att_dst1[0])
    Batt2 = jnp.zeros((128, 8), f32)
    Batt2 = Batt2.at[:OUT, 0].set(att_src2[0]).at[:OUT, 1].set(att_dst2[0])
    W2pad = jnp.zeros((HID, 128), f32).at[:, :OUT].set(W2)

    src = edge_index[0]
    dst = edge_index[1]

    h1, aux1 = _lin1(x, W1, Batt1)
    P1, den1 = _edge(src, dst, aux1[:, 0], aux1[:, 1], h1)
    h2, aux2 = _lin2(P1, den1.T, aux1, h1, b1, W2pad, Batt2)
    P2, den2 = _edge(src, dst, aux2[:, 0], aux2[:, 1], h2)
    return _final(P2, den2.T, aux2, h2, b2)
